# Initial kernel scaffold; baseline (speedup 1.0000x reference)
import jax
import jax.numpy as jnp
from jax.experimental import pallas as pl


def kernel(input_ids, embedding_weight):
    raise NotImplementedError("write your pallas kernel here")


# speedup vs baseline: 1.3366x; 1.3366x over previous
import functools

import jax
import jax.numpy as jnp
from jax import lax
from jax.experimental import pallas as pl
from jax.experimental.pallas import tpu as pltpu
from jax.experimental.pallas import tpu_sc as plsc

B, S, H, D = 4, 2048, 8, 128
HEAD_VOCAB = 100000
NC, NS, L = 2, 16, 16
NW = NC * NS
TOTAL = B * S * H
PER_W = TOTAL // NW
CH = 128
NCH = PER_W // CH


@functools.partial(
    pl.kernel,
    out_type=jax.ShapeDtypeStruct((NW, NCH, CH, D), jnp.float32),
    mesh=plsc.VectorSubcoreMesh(
        core_axis_name="c", subcore_axis_name="s",
        num_cores=NC, num_subcores=NS,
    ),
    scratch_types=[
        pltpu.VMEM((PER_W,), jnp.int32),
        pltpu.VMEM((CH, D), jnp.float32),
        pltpu.VMEM((CH, D), jnp.float32),
        pltpu.SemaphoreType.DMA,
    ],
)
def _engram_gather(idx_hbm, table_hbm, out_hbm, idx_v, rows0, rows1, gsem):
    wid = lax.axis_index("s") * NC + lax.axis_index("c")
    pltpu.sync_copy(idx_hbm.at[wid], idx_v)

    offs = (lax.iota(jnp.int32, L) & 7) * HEAD_VOCAB

    def add_offs(p, carry):
        sl = pl.ds(pl.multiple_of(p * L, L), L)
        idx_v[sl] = idx_v[sl] + offs
        return carry

    lax.fori_loop(0, PER_W // L, add_offs, 0)

    bufs = (rows0, rows1)

    def start(c, buf):
        return pltpu.async_copy(
            table_hbm.at[idx_v.at[pl.ds(c * CH, CH)]], buf, gsem)

    inflight = start(0, bufs[0])
    for c in range(NCH):
        inflight.wait()
        if c + 1 < NCH:
            nxt = start(c + 1, bufs[(c + 1) % 2])
        pltpu.sync_copy(bufs[c % 2], out_hbm.at[wid, c])
        if c + 1 < NCH:
            inflight = nxt


def kernel(input_ids, embedding_weight):
    idx = input_ids.reshape(NW, PER_W).astype(jnp.int32)
    out = _engram_gather(idx, embedding_weight)
    return out.reshape(B, S, H, D)


# speedup vs baseline: 1.5749x; 1.1783x over previous
import functools

import jax
import jax.numpy as jnp
from jax import lax
from jax.experimental import pallas as pl
from jax.experimental.pallas import tpu as pltpu
from jax.experimental.pallas import tpu_sc as plsc

B, S, H, D = 4, 2048, 8, 128
HEAD_VOCAB = 100000
NC, NS, L = 2, 16, 16
NW = NC * NS
TOTAL = B * S * H
PER_W = TOTAL // NW
CH = 128
NCH = PER_W // CH


NSLOT = 4


@functools.partial(
    pl.kernel,
    out_type=jax.ShapeDtypeStruct((NW, NCH, CH, D), jnp.float32),
    mesh=plsc.VectorSubcoreMesh(
        core_axis_name="c", subcore_axis_name="s",
        num_cores=NC, num_subcores=NS,
    ),
    scratch_types=[
        pltpu.VMEM((PER_W,), jnp.int32),
        [pltpu.VMEM((CH, D), jnp.float32) for _ in range(NSLOT)],
        [pltpu.SemaphoreType.DMA for _ in range(NSLOT)],
        [pltpu.SemaphoreType.DMA for _ in range(NSLOT)],
    ],
)
def _engram_gather(idx_hbm, table_hbm, out_hbm, idx_v, bufs, gsems, osems):
    wid = lax.axis_index("s") * NC + lax.axis_index("c")
    pltpu.sync_copy(idx_hbm.at[wid], idx_v)

    offs = (lax.iota(jnp.int32, L) & 7) * HEAD_VOCAB

    def shift_chunk(c):
        for p in range(CH // L):
            sl = pl.ds(c * CH + p * L, L)
            idx_v[sl] = idx_v[sl] + offs

    def start_gather(c):
        s = c % NSLOT
        return pltpu.async_copy(
            table_hbm.at[idx_v.at[pl.ds(c * CH, CH)]], bufs[s], gsems[s])

    gh = [None] * NCH
    oh = [None] * NCH
    for c in range(NSLOT - 1):
        shift_chunk(c)
        gh[c] = start_gather(c)
    for c in range(NCH):
        s = c % NSLOT
        gh[c].wait()
        n = c + NSLOT - 1
        if n < NCH:
            if c >= 1:
                oh[c - 1].wait()
            shift_chunk(n)
            gh[n] = start_gather(n)
        oh[c] = pltpu.async_copy(bufs[s], out_hbm.at[wid, c], osems[s])
    for c in range(NCH - NSLOT, NCH):
        oh[c].wait()


def kernel(input_ids, embedding_weight):
    idx = input_ids.reshape(NW, PER_W).astype(jnp.int32)
    out = _engram_gather(idx, embedding_weight)
    return out.reshape(B, S, H, D)


# speedup vs baseline: 1.6241x; 1.0312x over previous
import functools

import jax
import jax.numpy as jnp
from jax import lax
from jax.experimental import pallas as pl
from jax.experimental.pallas import tpu as pltpu
from jax.experimental.pallas import tpu_sc as plsc

B, S, H, D = 4, 2048, 8, 128
HEAD_VOCAB = 100000
NC, NS, L = 2, 16, 16
NW = NC * NS
TOTAL = B * S * H
PER_W = TOTAL // NW
CH = 128
NCH = PER_W // CH


NSLOT = 6


@functools.partial(
    pl.kernel,
    out_type=jax.ShapeDtypeStruct((NW, NCH, CH, D), jnp.float32),
    mesh=plsc.VectorSubcoreMesh(
        core_axis_name="c", subcore_axis_name="s",
        num_cores=NC, num_subcores=NS,
    ),
    scratch_types=[
        pltpu.VMEM((PER_W,), jnp.int32),
        [pltpu.VMEM((CH, D), jnp.float32) for _ in range(NSLOT)],
        [pltpu.SemaphoreType.DMA for _ in range(NSLOT)],
        [pltpu.SemaphoreType.DMA for _ in range(NSLOT)],
    ],
)
def _engram_gather(idx_hbm, table_hbm, out_hbm, idx_v, bufs, gsems, osems):
    wid = lax.axis_index("s") * NC + lax.axis_index("c")
    pltpu.sync_copy(idx_hbm.at[wid], idx_v)

    offs = (lax.iota(jnp.int32, L) & 7) * HEAD_VOCAB

    def shift_chunk(c):
        for p in range(CH // L):
            sl = pl.ds(c * CH + p * L, L)
            idx_v[sl] = idx_v[sl] + offs

    def start_gather(c):
        s = c % NSLOT
        return pltpu.async_copy(
            table_hbm.at[idx_v.at[pl.ds(c * CH, CH)]], bufs[s], gsems[s])

    gh = [None] * NCH
    oh = [None] * NCH
    for c in range(NSLOT - 1):
        shift_chunk(c)
        gh[c] = start_gather(c)
    for c in range(NCH):
        s = c % NSLOT
        gh[c].wait()
        n = c + NSLOT - 1
        if n < NCH:
            if c >= 1:
                oh[c - 1].wait()
            shift_chunk(n)
            gh[n] = start_gather(n)
        oh[c] = pltpu.async_copy(bufs[s], out_hbm.at[wid, c], osems[s])
    for c in range(NCH - NSLOT, NCH):
        oh[c].wait()


def kernel(input_ids, embedding_weight):
    idx = input_ids.reshape(NW, PER_W).astype(jnp.int32)
    out = _engram_gather(idx, embedding_weight)
    return out.reshape(B, S, H, D)


# speedup vs baseline: 1.6396x; 1.0095x over previous
import functools

import jax
import jax.numpy as jnp
from jax import lax
from jax.experimental import pallas as pl
from jax.experimental.pallas import tpu as pltpu
from jax.experimental.pallas import tpu_sc as plsc

B, S, H, D = 4, 2048, 8, 128
HEAD_VOCAB = 100000
NC, NS, L = 2, 16, 16
NW = NC * NS
TOTAL = B * S * H
PER_W = TOTAL // NW
CH = 128
NCH = PER_W // CH


NSLOT = 6


@functools.partial(
    pl.kernel,
    out_type=jax.ShapeDtypeStruct((NW, NCH, CH, D), jnp.float32),
    mesh=plsc.VectorSubcoreMesh(
        core_axis_name="c", subcore_axis_name="s",
        num_cores=NC, num_subcores=NS,
    ),
    scratch_types=[
        pltpu.VMEM((PER_W,), jnp.int32),
        [pltpu.VMEM((CH, D), jnp.float32) for _ in range(NSLOT)],
        [pltpu.SemaphoreType.DMA for _ in range(NSLOT)],
        [pltpu.SemaphoreType.DMA for _ in range(NSLOT)],
    ],
)
def _engram_gather(idx_hbm, table_hbm, out_hbm, idx_v, bufs, gsems, osems):
    wid = lax.axis_index("s") * NC + lax.axis_index("c")
    pltpu.sync_copy(idx_hbm.at[pl.ds(wid * PER_W, PER_W)], idx_v)

    offs = (lax.iota(jnp.int32, L) & 7) * HEAD_VOCAB

    def shift_chunk(c):
        for p in range(CH // L):
            sl = pl.ds(c * CH + p * L, L)
            idx_v[sl] = idx_v[sl] + offs

    def start_gather(c):
        s = c % NSLOT
        return pltpu.async_copy(
            table_hbm.at[idx_v.at[pl.ds(c * CH, CH)]], bufs[s], gsems[s])

    gh = [None] * NCH
    oh = [None] * NCH
    for c in range(NSLOT - 1):
        shift_chunk(c)
        gh[c] = start_gather(c)
    for c in range(NCH):
        s = c % NSLOT
        gh[c].wait()
        n = c + NSLOT - 1
        if n < NCH:
            if c >= 1:
                oh[c - 1].wait()
            shift_chunk(n)
            gh[n] = start_gather(n)
        oh[c] = pltpu.async_copy(bufs[s], out_hbm.at[wid, c], osems[s])
    for c in range(NCH - NSLOT, NCH):
        oh[c].wait()


def kernel(input_ids, embedding_weight):
    idx = input_ids.reshape(TOTAL).astype(jnp.int32)
    out = _engram_gather(idx, embedding_weight)
    return out.reshape(B, S, H, D)
